# baseline (speedup 1.0000x reference)
import jax
import jax.numpy as jnp
from jax.experimental import pallas as pl
from jax.experimental.pallas import tpu as pltpu

_N = 33554432
_BN = 1 << 20
_G = _N // _BN
_CH = 1 << 16
_K = _BN // _CH
_STASH = 18


def _softmax_body(x_ref, o_ref, acc_ref, st_ref, inv_ref):
    p = pl.program_id(0)
    i = pl.program_id(1)

    @pl.when((p == 0) & (i == 0))
    def _init():
        acc_ref[...] = jnp.zeros_like(acc_ref)

    @pl.when(p == 0)
    def _accumulate():
        for k in range(0, _K, 2):
            xa = x_ref[pl.ds(k * _CH, _CH)]
            xb = x_ref[pl.ds((k + 1) * _CH, _CH)]
            acc_ref[...] += jnp.exp(xa) + jnp.exp(xb)

        @pl.when(i < _STASH)
        def _stash():
            for k in range(_K):
                st_ref[pl.ds(i * _BN + k * _CH, _CH)] = (
                    x_ref[pl.ds(k * _CH, _CH)].astype(jnp.bfloat16))

    @pl.when((p == 1) & (i == 0))
    def _finalize():
        inv_ref[0] = 1.0 / jnp.sum(acc_ref[...])

    @pl.when(p == 1)
    def _scale():
        @pl.when(i < _STASH)
        def _from_stash():
            o_ref[...] = jnp.exp(
                st_ref[pl.ds(i * _BN, _BN)].astype(jnp.float32)) * inv_ref[0]

        @pl.when(i >= _STASH)
        def _from_hbm():
            o_ref[...] = jnp.exp(x_ref[...]) * inv_ref[0]


def kernel(x):
    return pl.pallas_call(
        _softmax_body,
        out_shape=jax.ShapeDtypeStruct((_N,), jnp.float32),
        grid=(2, _G),
        in_specs=[pl.BlockSpec(
            (_BN,),
            lambda p, i: (jnp.where((p == 1) & (i < _STASH), _G - 1, i),))],
        out_specs=pl.BlockSpec((_BN,), lambda p, i: (i * p,)),
        scratch_shapes=[
            pltpu.VMEM((_CH,), jnp.float32),
            pltpu.VMEM((_STASH * _BN,), jnp.bfloat16),
            pltpu.SMEM((1,), jnp.float32),
        ],
        compiler_params=pltpu.CompilerParams(
            dimension_semantics=("arbitrary", "arbitrary"),
            vmem_limit_bytes=56 * 1024 * 1024,
        ),
        name="flat_softmax",
    )(x)


# speedup vs baseline: 1.8640x; 1.8640x over previous
import jax
import jax.numpy as jnp
from jax.experimental import pallas as pl
from jax.experimental.pallas import tpu as pltpu

_N = 33554432
_BN = 1 << 20
_G = _N // _BN
_CH = 1 << 16
_K = _BN // _CH
_STASH = 9


def _softmax_body(x_ref, o_ref, acc_ref, st_ref, inv_ref):
    p = pl.program_id(0)
    i = pl.program_id(1)

    @pl.when((p == 0) & (i == 0))
    def _init():
        acc_ref[...] = jnp.zeros_like(acc_ref)

    @pl.when(p == 0)
    def _accumulate():
        for k in range(0, _K, 2):
            xa = x_ref[pl.ds(k * _CH, _CH)]
            xb = x_ref[pl.ds((k + 1) * _CH, _CH)]
            acc_ref[...] += jnp.exp(xa) + jnp.exp(xb)

        @pl.when(i < _STASH)
        def _stash():
            for k in range(_K):
                st_ref[pl.ds(i * _BN + k * _CH, _CH)] = x_ref[pl.ds(k * _CH, _CH)]

    @pl.when((p == 1) & (i == 0))
    def _finalize():
        inv_ref[0] = 1.0 / jnp.sum(acc_ref[...])

    @pl.when(p == 1)
    def _scale():
        @pl.when(i < _STASH)
        def _from_stash():
            o_ref[...] = jnp.exp(st_ref[pl.ds(i * _BN, _BN)]) * inv_ref[0]

        @pl.when(i >= _STASH)
        def _from_hbm():
            o_ref[...] = jnp.exp(x_ref[...]) * inv_ref[0]


def kernel(x):
    return pl.pallas_call(
        _softmax_body,
        out_shape=jax.ShapeDtypeStruct((_N,), jnp.float32),
        grid=(2, _G),
        in_specs=[pl.BlockSpec(
            (_BN,),
            lambda p, i: (jnp.where((p == 1) & (i < _STASH), _G - 1, i),))],
        out_specs=pl.BlockSpec((_BN,), lambda p, i: (i * p,)),
        scratch_shapes=[
            pltpu.VMEM((_CH,), jnp.float32),
            pltpu.VMEM((_STASH * _BN,), jnp.float32),
            pltpu.SMEM((1,), jnp.float32),
        ],
        compiler_params=pltpu.CompilerParams(
            dimension_semantics=("arbitrary", "arbitrary"),
            vmem_limit_bytes=56 * 1024 * 1024,
        ),
        name="flat_softmax",
    )(x)
